# baseline (speedup 1.0000x reference)
import functools

import jax
import jax.numpy as jnp
from jax import lax
from jax.experimental import pallas as pl
from jax.experimental.pallas import tpu as pltpu
from jax.experimental.pallas import tpu_sc as plsc

N, E, D_IN, D_H, D_OUT = 10000, 320000, 128, 64, 128

NC, NS = 2, 16
NW = NC * NS
EB = 128
K = 4
NBLK = 21
EPT = EB * K * NBLK
EPAD = EPT * NW
NBATCH = EPT // EB
NPAD = 10240
RPT = NPAD // NS
RCH = 128
CW = 16
SHIFT = 14
MASK = (1 << SHIFT) - 1

_f32 = jnp.float32


def _seg_sum_kernel(with_cnt):
    mesh = plsc.VectorSubcoreMesh(
        core_axis_name="c", subcore_axis_name="s",
        num_cores=NC, num_subcores=NS)

    out_type = [jax.ShapeDtypeStruct((NC, NPAD, D_H), _f32)]
    scratch = [
        pltpu.VMEM((K, EB), jnp.int32),
        pltpu.VMEM((K, EB), jnp.int32),
        pltpu.VMEM((K, EB, D_H), _f32),
        pltpu.VMEM((RCH, D_H), _f32),
        pltpu.VMEM_SHARED((NPAD, D_H), _f32),
    ]
    if with_cnt:
        out_type.append(jax.ShapeDtypeStruct((NC, NPAD, CW), _f32))
        scratch += [
            pltpu.VMEM((EB, CW), _f32),
            pltpu.VMEM((RCH, CW), _f32),
            pltpu.VMEM_SHARED((NPAD, CW), _f32),
        ]

    @functools.partial(pl.kernel, mesh=mesh, out_type=out_type,
                       scratch_types=scratch,
                       compiler_params=pltpu.CompilerParams(
                           use_tc_tiling_on_sc=False))
    def body(table_hbm, src_hbm, dst_hbm, *refs):
        if with_cnt:
            (sum_out, cnt_out, src_v, dst_v, rows_v, zb_v, acc_sh,
             ones_v, zc_v, cnt_sh) = refs
        else:
            (sum_out, src_v, dst_v, rows_v, zb_v, acc_sh) = refs
        c = lax.axis_index("c")
        s = lax.axis_index("s")
        wid = s * NC + c

        zeros16 = jnp.zeros((16,), _f32)

        def zero_row(i, _):
            for j in range(D_H // 16):
                zb_v[i, pl.ds(16 * j, 16)] = zeros16
            if with_cnt:
                zc_v[i, pl.ds(0, 16)] = zeros16
            return 0

        lax.fori_loop(0, RCH, zero_row, 0)
        if with_cnt:
            def one_row(i, _):
                ones_v[i, pl.ds(0, 16)] = zeros16 + 1.0
                return 0
            lax.fori_loop(0, EB, one_row, 0)

        row0 = s * RPT
        for k in range(RPT // RCH):
            pltpu.sync_copy(zb_v, acc_sh.at[pl.ds(row0 + k * RCH, RCH)])
            if with_cnt:
                pltpu.sync_copy(zc_v, cnt_sh.at[pl.ds(row0 + k * RCH, RCH)])
        plsc.subcore_barrier()

        def blk(j, _):
            rb = wid * NBATCH + j * K
            pltpu.sync_copy(src_hbm.at[pl.ds(rb, K)], src_v)
            pltpu.sync_copy(dst_hbm.at[pl.ds(rb, K)], dst_v)
            for k in range(K):
                pltpu.sync_copy(table_hbm.at[src_v.at[k]], rows_v.at[k])
            for k in range(K):
                pltpu.sync_copy(rows_v.at[k], acc_sh.at[dst_v.at[k]],
                                add=True)
                if with_cnt:
                    pltpu.sync_copy(ones_v, cnt_sh.at[dst_v.at[k]], add=True)
            return 0

        lax.fori_loop(0, NBLK, blk, 0)

        plsc.subcore_barrier()

        for k in range(RPT // RCH):
            r = row0 + k * RCH
            pltpu.sync_copy(acc_sh.at[pl.ds(r, RCH)], sum_out.at[c, pl.ds(r, RCH)])
            if with_cnt:
                pltpu.sync_copy(cnt_sh.at[pl.ds(r, RCH)], cnt_out.at[c, pl.ds(r, RCH)])

    return body


_seg_sum_cnt = _seg_sum_kernel(True)
_seg_sum = _seg_sum_kernel(False)


def _tc_proj1(x_ref, w_ref, b_ref, y_ref, z_ref):
    yz = jnp.dot(x_ref[...], w_ref[...], preferred_element_type=_f32)
    y_ref[...] = yz[:, :D_H]
    z_ref[...] = yz[:, D_H:] + b_ref[...]


def _tc_mid(parts_ref, cnts_ref, z_ref, h_ref):
    agg = parts_ref[0, :N] + parts_ref[1, :N]
    cnt = cnts_ref[0, :N, 0:1] + cnts_ref[1, :N, 0:1]
    hpre = agg / jnp.maximum(cnt, 1.0) + z_ref[...]
    col_mean = jnp.mean(hpre, axis=0, keepdims=True)
    rn = jnp.sqrt(1e-6 + jnp.sum(hpre * hpre, axis=1, keepdims=True))
    h_ref[...] = jnp.maximum(hpre / rn - col_mean, 0.0)


def _tc_out(parts_ref, cnts_ref, h_ref, wn_ref, wr_ref, b_ref, o_ref):
    agg = parts_ref[0, :N] + parts_ref[1, :N]
    cnt = cnts_ref[0, :N, 0:1] + cnts_ref[1, :N, 0:1]
    mean = agg / jnp.maximum(cnt, 1.0)
    o_ref[...] = (jnp.dot(mean, wn_ref[...], preferred_element_type=_f32)
                  + jnp.dot(h_ref[...], wr_ref[...], preferred_element_type=_f32)
                  + b_ref[...])


def kernel(x, edge_index, Wn1, Wr1, b1, Wn2, Wr2, b2):
    pad = EPAD - E
    src2d = jnp.concatenate(
        [edge_index[0], jnp.zeros((pad,), jnp.int32)]).reshape(EPAD // EB, EB)
    pad_dst = N + jnp.arange(pad, dtype=jnp.int32) % (NPAD - N)
    dst2d = jnp.concatenate(
        [edge_index[1], pad_dst]).reshape(EPAD // EB, EB)

    w1 = jnp.concatenate([Wn1, Wr1], axis=1)
    bias1 = b1[None, :]

    y1, z1 = pl.pallas_call(
        _tc_proj1,
        out_shape=[jax.ShapeDtypeStruct((N, D_H), _f32),
                   jax.ShapeDtypeStruct((N, D_H), _f32)],
    )(x, w1, bias1)

    parts1, cnts = _seg_sum_cnt(y1, src2d, dst2d)

    h = pl.pallas_call(
        _tc_mid,
        out_shape=jax.ShapeDtypeStruct((N, D_H), _f32),
    )(parts1, cnts, z1)

    parts2, = _seg_sum(h, src2d, dst2d)

    out = pl.pallas_call(
        _tc_out,
        out_shape=jax.ShapeDtypeStruct((N, D_OUT), _f32),
    )(parts2, cnts, h, Wn2, Wr2, b2[None, :])
    return out


# speedup vs baseline: 1.2973x; 1.2973x over previous
import functools

import jax
import jax.numpy as jnp
from jax import lax
from jax.experimental import pallas as pl
from jax.experimental.pallas import tpu as pltpu
from jax.experimental.pallas import tpu_sc as plsc

N, E, D_IN, D_H, D_OUT = 10000, 320000, 128, 64, 128

NC, NS = 2, 16
NW = NC * NS
EB = 80
K = 5
NBLK = 26
EPT = EB * K * NBLK
EPAD = EPT * NW
NBATCH = EPT // EB
NPAD = 10240
RPT = NPAD // NS
RCH = 128
CW = 16
SHIFT = 14
MASK = (1 << SHIFT) - 1

_f32 = jnp.float32


def _seg_sum_kernel(with_cnt):
    mesh = plsc.VectorSubcoreMesh(
        core_axis_name="c", subcore_axis_name="s",
        num_cores=NC, num_subcores=NS)

    out_type = [jax.ShapeDtypeStruct((NC, NPAD, D_H), _f32)]
    scratch = [
        pltpu.VMEM((K, EB), jnp.int32),
        pltpu.VMEM((K, EB), jnp.int32),
        pltpu.VMEM((K, EB, D_H), _f32),
        pltpu.VMEM((RCH, D_H), _f32),
        pltpu.VMEM_SHARED((NPAD, D_H), _f32),
    ]
    if with_cnt:
        out_type.append(jax.ShapeDtypeStruct((NC, NPAD, CW), _f32))
        scratch += [
            pltpu.VMEM((EB, CW), _f32),
            pltpu.VMEM((RCH, CW), _f32),
            pltpu.VMEM_SHARED((NPAD, CW), _f32),
        ]

    @functools.partial(pl.kernel, mesh=mesh, out_type=out_type,
                       scratch_types=scratch,
                       compiler_params=pltpu.CompilerParams(
                           use_tc_tiling_on_sc=False))
    def body(table_hbm, src_hbm, dst_hbm, *refs):
        if with_cnt:
            (sum_out, cnt_out, src_v, dst_v, rows_v, zb_v, acc_sh,
             ones_v, zc_v, cnt_sh) = refs
        else:
            (sum_out, src_v, dst_v, rows_v, zb_v, acc_sh) = refs
        c = lax.axis_index("c")
        s = lax.axis_index("s")
        wid = s * NC + c

        zeros16 = jnp.zeros((16,), _f32)

        def zero_row(i, _):
            for j in range(D_H // 16):
                zb_v[i, pl.ds(16 * j, 16)] = zeros16
            if with_cnt:
                zc_v[i, pl.ds(0, 16)] = zeros16
            return 0

        lax.fori_loop(0, RCH, zero_row, 0)
        if with_cnt:
            def one_row(i, _):
                ones_v[i, pl.ds(0, 16)] = zeros16 + 1.0
                return 0
            lax.fori_loop(0, EB, one_row, 0)

        row0 = s * RPT
        for k in range(RPT // RCH):
            pltpu.sync_copy(zb_v, acc_sh.at[pl.ds(row0 + k * RCH, RCH)])
            if with_cnt:
                pltpu.sync_copy(zc_v, cnt_sh.at[pl.ds(row0 + k * RCH, RCH)])
        plsc.subcore_barrier()

        def blk(j, _):
            rb = wid * NBATCH + j * K
            pltpu.sync_copy(src_hbm.at[pl.ds(rb, K)], src_v)
            pltpu.sync_copy(dst_hbm.at[pl.ds(rb, K)], dst_v)
            for k in range(K):
                pltpu.sync_copy(table_hbm.at[src_v.at[k]], rows_v.at[k])
            for k in range(K):
                pltpu.sync_copy(rows_v.at[k], acc_sh.at[dst_v.at[k]],
                                add=True)
                if with_cnt:
                    pltpu.sync_copy(ones_v, cnt_sh.at[dst_v.at[k]], add=True)
            return 0

        lax.fori_loop(0, NBLK, blk, 0)

        plsc.subcore_barrier()

        for k in range(RPT // RCH):
            r = row0 + k * RCH
            pltpu.sync_copy(acc_sh.at[pl.ds(r, RCH)], sum_out.at[c, pl.ds(r, RCH)])
            if with_cnt:
                pltpu.sync_copy(cnt_sh.at[pl.ds(r, RCH)], cnt_out.at[c, pl.ds(r, RCH)])

    return body


_seg_sum_cnt = _seg_sum_kernel(True)
_seg_sum = _seg_sum_kernel(False)


def _tc_proj1(x_ref, w_ref, b_ref, y_ref, z_ref):
    yz = jnp.dot(x_ref[...], w_ref[...], preferred_element_type=_f32)
    y_ref[...] = yz[:, :D_H]
    z_ref[...] = yz[:, D_H:] + b_ref[...]


def _tc_mid(parts_ref, cnts_ref, z_ref, h_ref):
    agg = parts_ref[0, :N] + parts_ref[1, :N]
    cnt = cnts_ref[0, :N, 0:1] + cnts_ref[1, :N, 0:1]
    hpre = agg / jnp.maximum(cnt, 1.0) + z_ref[...]
    col_mean = jnp.mean(hpre, axis=0, keepdims=True)
    rn = jnp.sqrt(1e-6 + jnp.sum(hpre * hpre, axis=1, keepdims=True))
    h_ref[...] = jnp.maximum(hpre / rn - col_mean, 0.0)


def _tc_out(parts_ref, cnts_ref, h_ref, wn_ref, wr_ref, b_ref, o_ref):
    agg = parts_ref[0, :N] + parts_ref[1, :N]
    cnt = cnts_ref[0, :N, 0:1] + cnts_ref[1, :N, 0:1]
    mean = agg / jnp.maximum(cnt, 1.0)
    o_ref[...] = (jnp.dot(mean, wn_ref[...], preferred_element_type=_f32)
                  + jnp.dot(h_ref[...], wr_ref[...], preferred_element_type=_f32)
                  + b_ref[...])


def kernel(x, edge_index, Wn1, Wr1, b1, Wn2, Wr2, b2):
    pad = EPAD - E
    src2d = jnp.concatenate(
        [edge_index[0], jnp.zeros((pad,), jnp.int32)]).reshape(EPAD // EB, EB)
    pad_dst = N + jnp.arange(pad, dtype=jnp.int32) % (NPAD - N)
    dst2d = jnp.concatenate(
        [edge_index[1], pad_dst]).reshape(EPAD // EB, EB)

    w1 = jnp.concatenate([Wn1, Wr1], axis=1)
    bias1 = b1[None, :]

    y1, z1 = pl.pallas_call(
        _tc_proj1,
        out_shape=[jax.ShapeDtypeStruct((N, D_H), _f32),
                   jax.ShapeDtypeStruct((N, D_H), _f32)],
    )(x, w1, bias1)

    parts1, cnts = _seg_sum_cnt(y1, src2d, dst2d)

    h = pl.pallas_call(
        _tc_mid,
        out_shape=jax.ShapeDtypeStruct((N, D_H), _f32),
    )(parts1, cnts, z1)

    parts2, = _seg_sum(h, src2d, dst2d)

    out = pl.pallas_call(
        _tc_out,
        out_shape=jax.ShapeDtypeStruct((N, D_OUT), _f32),
    )(parts2, cnts, h, Wn2, Wr2, b2[None, :])
    return out


# speedup vs baseline: 3.6550x; 2.8174x over previous
import functools

import jax
import jax.numpy as jnp
from jax import lax
from jax.experimental import pallas as pl
from jax.experimental.pallas import tpu as pltpu
from jax.experimental.pallas import tpu_sc as plsc

N, E, D_IN, D_H, D_OUT = 10000, 320000, 128, 64, 128

NC, NS = 2, 16
NW = NC * NS
EB = 400
K = 1
NBLK = 25
EPT = EB * K * NBLK
EPAD = EPT * NW
NBATCH = EPT // EB
NPAD = 10240
RPT = NPAD // NS
RCH = 128
CW = 16
SHIFT = 14
MASK = (1 << SHIFT) - 1

_f32 = jnp.float32


def _seg_sum_kernel(with_cnt):
    mesh = plsc.VectorSubcoreMesh(
        core_axis_name="c", subcore_axis_name="s",
        num_cores=NC, num_subcores=NS)

    out_type = [jax.ShapeDtypeStruct((NC, NPAD, D_H), _f32)]
    scratch = [
        pltpu.VMEM((K, EB), jnp.int32),
        pltpu.VMEM((K, EB), jnp.int32),
        pltpu.VMEM((K, EB, D_H), _f32),
        pltpu.VMEM((RCH, D_H), _f32),
        pltpu.VMEM_SHARED((NPAD, D_H), _f32),
    ]
    if with_cnt:
        out_type.append(jax.ShapeDtypeStruct((NC, NPAD, CW), _f32))
        scratch += [
            pltpu.VMEM((EB, CW), _f32),
            pltpu.VMEM((RCH, CW), _f32),
            pltpu.VMEM_SHARED((NPAD, CW), _f32),
        ]

    @functools.partial(pl.kernel, mesh=mesh, out_type=out_type,
                       scratch_types=scratch,
                       compiler_params=pltpu.CompilerParams(
                           use_tc_tiling_on_sc=False))
    def body(table_hbm, src_hbm, dst_hbm, *refs):
        if with_cnt:
            (sum_out, cnt_out, src_v, dst_v, rows_v, zb_v, acc_sh,
             ones_v, zc_v, cnt_sh) = refs
        else:
            (sum_out, src_v, dst_v, rows_v, zb_v, acc_sh) = refs
        c = lax.axis_index("c")
        s = lax.axis_index("s")
        wid = s * NC + c

        zeros16 = jnp.zeros((16,), _f32)

        def zero_row(i, _):
            for j in range(D_H // 16):
                zb_v[i, pl.ds(16 * j, 16)] = zeros16
            if with_cnt:
                zc_v[i, pl.ds(0, 16)] = zeros16
            return 0

        lax.fori_loop(0, RCH, zero_row, 0)
        if with_cnt:
            def one_row(i, _):
                ones_v[i, pl.ds(0, 16)] = zeros16 + 1.0
                return 0
            lax.fori_loop(0, EB, one_row, 0)

        row0 = s * RPT
        for k in range(RPT // RCH):
            pltpu.sync_copy(zb_v, acc_sh.at[pl.ds(row0 + k * RCH, RCH)])
            if with_cnt:
                pltpu.sync_copy(zc_v, cnt_sh.at[pl.ds(row0 + k * RCH, RCH)])
        plsc.subcore_barrier()

        def blk(j, _):
            rb = wid * NBATCH + j * K
            pltpu.sync_copy(src_hbm.at[pl.ds(rb, K)], src_v)
            pltpu.sync_copy(dst_hbm.at[pl.ds(rb, K)], dst_v)
            for k in range(K):
                pltpu.sync_copy(table_hbm.at[src_v.at[k]], rows_v.at[k])
            for k in range(K):
                pltpu.sync_copy(rows_v.at[k], acc_sh.at[dst_v.at[k]],
                                add=True)
                if with_cnt:
                    pltpu.sync_copy(ones_v, cnt_sh.at[dst_v.at[k]], add=True)
            return 0

        lax.fori_loop(0, NBLK, blk, 0)

        plsc.subcore_barrier()

        for k in range(RPT // RCH):
            r = row0 + k * RCH
            pltpu.sync_copy(acc_sh.at[pl.ds(r, RCH)], sum_out.at[c, pl.ds(r, RCH)])
            if with_cnt:
                pltpu.sync_copy(cnt_sh.at[pl.ds(r, RCH)], cnt_out.at[c, pl.ds(r, RCH)])

    return body


_seg_sum_cnt = _seg_sum_kernel(True)
_seg_sum = _seg_sum_kernel(False)


def _tc_proj1(x_ref, w_ref, b_ref, y_ref, z_ref):
    yz = jnp.dot(x_ref[...], w_ref[...], preferred_element_type=_f32)
    y_ref[...] = yz[:, :D_H]
    z_ref[...] = yz[:, D_H:] + b_ref[...]


def _tc_mid(parts_ref, cnts_ref, z_ref, h_ref):
    agg = parts_ref[0, :N] + parts_ref[1, :N]
    cnt = cnts_ref[0, :N, 0:1] + cnts_ref[1, :N, 0:1]
    hpre = agg / jnp.maximum(cnt, 1.0) + z_ref[...]
    col_mean = jnp.mean(hpre, axis=0, keepdims=True)
    rn = jnp.sqrt(1e-6 + jnp.sum(hpre * hpre, axis=1, keepdims=True))
    h_ref[...] = jnp.maximum(hpre / rn - col_mean, 0.0)


def _tc_out(parts_ref, cnts_ref, h_ref, wn_ref, wr_ref, b_ref, o_ref):
    agg = parts_ref[0, :N] + parts_ref[1, :N]
    cnt = cnts_ref[0, :N, 0:1] + cnts_ref[1, :N, 0:1]
    mean = agg / jnp.maximum(cnt, 1.0)
    o_ref[...] = (jnp.dot(mean, wn_ref[...], preferred_element_type=_f32)
                  + jnp.dot(h_ref[...], wr_ref[...], preferred_element_type=_f32)
                  + b_ref[...])


def kernel(x, edge_index, Wn1, Wr1, b1, Wn2, Wr2, b2):
    pad = EPAD - E
    src2d = jnp.concatenate(
        [edge_index[0], jnp.zeros((pad,), jnp.int32)]).reshape(EPAD // EB, EB)
    pad_dst = N + jnp.arange(pad, dtype=jnp.int32) % (NPAD - N)
    dst2d = jnp.concatenate(
        [edge_index[1], pad_dst]).reshape(EPAD // EB, EB)

    w1 = jnp.concatenate([Wn1, Wr1], axis=1)
    bias1 = b1[None, :]

    y1, z1 = pl.pallas_call(
        _tc_proj1,
        out_shape=[jax.ShapeDtypeStruct((N, D_H), _f32),
                   jax.ShapeDtypeStruct((N, D_H), _f32)],
    )(x, w1, bias1)

    parts1, cnts = _seg_sum_cnt(y1, src2d, dst2d)

    h = pl.pallas_call(
        _tc_mid,
        out_shape=jax.ShapeDtypeStruct((N, D_H), _f32),
    )(parts1, cnts, z1)

    parts2, = _seg_sum(h, src2d, dst2d)

    out = pl.pallas_call(
        _tc_out,
        out_shape=jax.ShapeDtypeStruct((N, D_OUT), _f32),
    )(parts2, cnts, h, Wn2, Wr2, b2[None, :])
    return out
